# baseline (speedup 1.0000x reference)
import functools

import jax
import jax.numpy as jnp
from jax import lax
from jax.experimental import pallas as pl
from jax.experimental.pallas import tpu as pltpu
from jax.experimental.pallas import tpu_sc as plsc

L = 16
NC = 2
NS = 16
NW = NC * NS
CB = 128


def kernel(features, state, batch):
    N, D = features.shape
    G = state.shape[0]
    CPW = ((N + NW - 1) // NW + L - 1) // L * L
    C = 112
    NCHUNK = CPW // C
    NBUF = 7
    assert NCHUNK % NBUF == 0

    state_flat = state.reshape(G)

    mesh = plsc.VectorSubcoreMesh(core_axis_name="c", subcore_axis_name="s")

    @functools.partial(
        pl.kernel,
        mesh=mesh,
        compiler_params=pltpu.CompilerParams(
            needs_layout_passes=False,
            skip_device_barrier=True,
            disable_bounds_checks=True,
            disable_semaphore_checks=True,
        ),
        out_type=(
            jax.ShapeDtypeStruct((N,), jnp.float32),
            jax.ShapeDtypeStruct((N,), jnp.float32),
            jax.ShapeDtypeStruct((N,), jnp.float32),
            jax.ShapeDtypeStruct((N,), jnp.float32),
        ),
        scratch_types=(
            [pltpu.VMEM((C, CB), jnp.float32)] * NBUF
            + [
                pltpu.VMEM((CPW,), jnp.int32),
                pltpu.VMEM((G,), jnp.int32),
                pltpu.VMEM((CPW,), jnp.float32),
                pltpu.VMEM((CPW,), jnp.float32),
                pltpu.VMEM((CPW,), jnp.float32),
                pltpu.VMEM((CPW,), jnp.float32),
            ]
            + [pltpu.SemaphoreType.DMA] * NBUF
        ),
    )
    def sc_kernel(feat_hbm, state_hbm, batch_hbm, e_hbm, x_hbm, y_hbm, z_hbm,
                  *scratch):
        fbufs = scratch[:NBUF]
        batch_v, state_v, e_v, x_v, y_v, z_v = scratch[NBUF:NBUF + 6]
        sems = scratch[NBUF + 6:]
        wid = lax.axis_index("s") * NC + lax.axis_index("c")
        base = jnp.minimum(wid * CPW, N - CPW)

        def feat_dma(t, b):
            return pltpu.make_async_copy(
                feat_hbm.at[pl.ds(base + t * C, C), pl.ds(0, CB)],
                fbufs[b], sems[b])

        for t in range(NBUF - 1):
            feat_dma(t, t).start()
        pltpu.sync_copy(state_hbm, state_v)
        pltpu.sync_copy(batch_hbm.at[pl.ds(base, CPW)], batch_v)

        iota = lax.iota(jnp.int32, L)
        c2 = jnp.full((L,), 2, jnp.int32)
        c3 = jnp.full((L,), 3, jnp.int32)
        c4 = jnp.full((L,), 4, jnp.int32)

        def outer(k, carry):
            t0 = k * NBUF
            for b in range(NBUF):
                t = t0 + b
                nxt = t + NBUF - 1

                @pl.when(nxt < NCHUNK)
                def _(nxt=nxt, b=b):
                    feat_dma(nxt, (b + NBUF - 1) % NBUF).start()

                feat_dma(t, b).wait()
                fb = fbufs[b]

                def body(j, carry2, t=t, fb=fb):
                    n = iota + j * L
                    g = batch_v[pl.ds(t * C + j * L, L)]
                    s = plsc.load_gather(state_v, [g])
                    e = plsc.load_gather(fb, [n, s])
                    x = plsc.load_gather(fb, [n, c4])
                    y = plsc.load_gather(fb, [n, c2])
                    z = plsc.load_gather(fb, [n, c3])
                    o = pl.ds(t * C + j * L, L)
                    e_v[o] = e
                    x_v[o] = x
                    y_v[o] = y
                    z_v[o] = z
                    return carry2

                lax.fori_loop(0, C // L, body, 0)
            return carry

        lax.fori_loop(0, NCHUNK // NBUF, outer, 0)

        pltpu.sync_copy(e_v, e_hbm.at[pl.ds(base, CPW)])
        pltpu.sync_copy(x_v, x_hbm.at[pl.ds(base, CPW)])
        pltpu.sync_copy(y_v, y_hbm.at[pl.ds(base, CPW)])
        pltpu.sync_copy(z_v, z_hbm.at[pl.ds(base, CPW)])

    e, x, y, z = sc_kernel(features, state_flat, batch)
    return e.reshape(N, 1), jnp.stack([x, y, z], axis=-1)


# speedup vs baseline: 1.0198x; 1.0198x over previous
import functools

import jax
import jax.numpy as jnp
from jax import lax
from jax.experimental import pallas as pl
from jax.experimental.pallas import tpu as pltpu
from jax.experimental.pallas import tpu_sc as plsc

L = 16
NC = 2
NS = 16
NW = NC * NS
CB = 128


def kernel(features, state, batch):
    N, D = features.shape
    G = state.shape[0]
    CPW = ((N + NW - 1) // NW + L - 1) // L * L
    C = 112
    NCHUNK = CPW // C
    NBUF = 4
    assert NCHUNK % NBUF == 0

    state_flat = state.reshape(G)

    mesh = plsc.VectorSubcoreMesh(core_axis_name="c", subcore_axis_name="s")

    @functools.partial(
        pl.kernel,
        mesh=mesh,
        compiler_params=pltpu.CompilerParams(
            needs_layout_passes=False,
            skip_device_barrier=True,
            disable_bounds_checks=True,
            disable_semaphore_checks=True,
        ),
        out_type=(
            jax.ShapeDtypeStruct((N,), jnp.float32),
            jax.ShapeDtypeStruct((N,), jnp.float32),
            jax.ShapeDtypeStruct((N,), jnp.float32),
            jax.ShapeDtypeStruct((N,), jnp.float32),
        ),
        scratch_types=(
            [pltpu.VMEM((C, CB), jnp.float32)] * NBUF
            + [
                pltpu.VMEM((CPW,), jnp.int32),
                pltpu.VMEM((G,), jnp.int32),
                pltpu.VMEM((CPW,), jnp.float32),
                pltpu.VMEM((CPW,), jnp.float32),
                pltpu.VMEM((CPW,), jnp.float32),
                pltpu.VMEM((CPW,), jnp.float32),
            ]
            + [pltpu.SemaphoreType.DMA] * (NBUF + 2)
        ),
    )
    def sc_kernel(feat_hbm, state_hbm, batch_hbm, e_hbm, x_hbm, y_hbm, z_hbm,
                  *scratch):
        fbufs = scratch[:NBUF]
        batch_v, state_v, e_v, x_v, y_v, z_v = scratch[NBUF:NBUF + 6]
        sems = scratch[NBUF + 6:NBUF + 6 + NBUF]
        sem_in, sem_out = scratch[NBUF + 6 + NBUF:]
        wid = lax.axis_index("s") * NC + lax.axis_index("c")
        base = jnp.minimum(wid * CPW, N - CPW)

        def feat_dma(t, b):
            return pltpu.make_async_copy(
                feat_hbm.at[pl.ds(base + t * C, C), pl.ds(0, CB)],
                fbufs[b], sems[b])

        cp_state = pltpu.make_async_copy(state_hbm, state_v, sem_in)
        cp_batch = pltpu.make_async_copy(
            batch_hbm.at[pl.ds(base, CPW)], batch_v, sem_in)
        cp_state.start()
        cp_batch.start()
        for t in range(NBUF - 1):
            feat_dma(t, t).start()
        cp_state.wait()
        cp_batch.wait()

        iota = lax.iota(jnp.int32, L)
        c2 = jnp.full((L,), 2, jnp.int32)
        c3 = jnp.full((L,), 3, jnp.int32)
        c4 = jnp.full((L,), 4, jnp.int32)

        def outer(k, carry):
            t0 = k * NBUF
            for b in range(NBUF):
                t = t0 + b
                nxt = t + NBUF - 1

                @pl.when(nxt < NCHUNK)
                def _(nxt=nxt, b=b):
                    feat_dma(nxt, (b + NBUF - 1) % NBUF).start()

                feat_dma(t, b).wait()
                fb = fbufs[b]

                def body(j, carry2, t=t, fb=fb):
                    n = iota + j * L
                    g = batch_v[pl.ds(t * C + j * L, L)]
                    s = plsc.load_gather(state_v, [g])
                    e = plsc.load_gather(fb, [n, s])
                    x = plsc.load_gather(fb, [n, c4])
                    y = plsc.load_gather(fb, [n, c2])
                    z = plsc.load_gather(fb, [n, c3])
                    o = pl.ds(t * C + j * L, L)
                    e_v[o] = e
                    x_v[o] = x
                    y_v[o] = y
                    z_v[o] = z
                    return carry2

                lax.fori_loop(0, C // L, body, 0)
            return carry

        lax.fori_loop(0, NCHUNK // NBUF, outer, 0)

        outs = [
            pltpu.make_async_copy(v, h.at[pl.ds(base, CPW)], sem_out)
            for v, h in ((e_v, e_hbm), (x_v, x_hbm), (y_v, y_hbm), (z_v, z_hbm))
        ]
        for cp in outs:
            cp.start()
        for cp in outs:
            cp.wait()

    e, x, y, z = sc_kernel(features, state_flat, batch)
    return e.reshape(N, 1), jnp.stack([x, y, z], axis=-1)
